# baseline (speedup 1.0000x reference)
import functools

import jax
import jax.numpy as jnp
from jax import lax
from jax.experimental import pallas as pl
from jax.experimental.pallas import tpu as pltpu
from jax.experimental.pallas import tpu_sc as plsc

_NC = 2
_NS = 16
_NW = _NC * _NS
_L = 16
_B = 8192
_D = 64
_BPW = _B // _NW
_GCH = 128
_NG = _BPW // _GCH

_mesh = plsc.VectorSubcoreMesh(core_axis_name="c", subcore_axis_name="s")


@functools.partial(
    pl.kernel,
    mesh=_mesh,
    compiler_params=pltpu.CompilerParams(use_tc_tiling_on_sc=False),
    out_type=(
        jax.ShapeDtypeStruct((_NW, _L), jnp.float32),
        jax.ShapeDtypeStruct((_NW, _L), jnp.float32),
        jax.ShapeDtypeStruct((_NW, _L), jnp.float32),
    ),
    scratch_types=[
        pltpu.VMEM((_NG, _GCH), jnp.int32),
        pltpu.VMEM((_BPW, _D), jnp.float32),
        pltpu.VMEM((_BPW, _D), jnp.float32),
        pltpu.VMEM((3, _L), jnp.float32),
        pltpu.SemaphoreType.DMA,
    ],
)
def _sc_partials(idx_hbm, feat_hbm, table_hbm, e2_hbm, dt_hbm, f2_hbm,
                 idx_v, rows_v, feat_v, acc_v, sem):
    wid = lax.axis_index("s") * _NC + lax.axis_index("c")
    base = wid * _BPW
    pltpu.sync_copy(idx_hbm.at[pl.ds(wid * _NG, _NG)], idx_v)
    copies = [
        pltpu.async_copy(
            table_hbm.at[idx_v.at[g]],
            rows_v.at[pl.ds(g * _GCH, _GCH)],
            sem,
        )
        for g in range(_NG)
    ]
    pltpu.sync_copy(feat_hbm.at[pl.ds(base, _BPW)], feat_v)
    for c in copies:
        c.wait()

    zeros = jnp.zeros((_L,), jnp.float32)

    def body(i, carry):
        e2, dt, f2 = carry
        for j in range(_D // _L):
            r = rows_v[i, pl.ds(j * _L, _L)]
            f = feat_v[i, pl.ds(j * _L, _L)]
            e2 = e2 + r * r
            dt = dt + r * f
            f2 = f2 + f * f
        return (e2, dt, f2)

    e2, dt, f2 = lax.fori_loop(0, _BPW, body, (zeros, zeros, zeros))
    acc_v[0, :] = e2
    acc_v[1, :] = dt
    acc_v[2, :] = f2
    pltpu.sync_copy(acc_v.at[0], e2_hbm.at[wid])
    pltpu.sync_copy(acc_v.at[1], dt_hbm.at[wid])
    pltpu.sync_copy(acc_v.at[2], f2_hbm.at[wid])


def _combine_body(e2_ref, dt_ref, f2_ref, o_ref):
    e2 = jnp.sum(e2_ref[...])
    dt = jnp.sum(dt_ref[...])
    f2 = jnp.sum(f2_ref[...])
    o_ref[0, 0] = 2.0 * f2 - 2.0 * jnp.sqrt(f2 / e2) * dt


_combine = pl.pallas_call(
    _combine_body,
    out_shape=jax.ShapeDtypeStruct((1, 1), jnp.float32),
    out_specs=pl.BlockSpec(memory_space=pltpu.SMEM),
)


def kernel(feature, item, neigh, pca_emb):
    idx = jnp.concatenate([item, neigh]).astype(jnp.int32)
    idx2d = idx.reshape(_NW * _NG, _GCH)
    e2p, dtp, f2p = _sc_partials(idx2d, feature, pca_emb)
    out = _combine(e2p, dtp, f2p)
    return out[0, 0]


# speedup vs baseline: 1.3968x; 1.3968x over previous
import functools

import jax
import jax.numpy as jnp
from jax import lax
from jax.experimental import pallas as pl
from jax.experimental.pallas import tpu as pltpu
from jax.experimental.pallas import tpu_sc as plsc

_NC = 2
_NS = 16
_NW = _NC * _NS
_L = 16
_B = 8192
_D = 64
_BPW = _B // _NW

_mesh = plsc.VectorSubcoreMesh(core_axis_name="c", subcore_axis_name="s")


@functools.partial(
    pl.kernel,
    mesh=_mesh,
    out_type=(
        jax.ShapeDtypeStruct((_NW, _L), jnp.float32),
        jax.ShapeDtypeStruct((_NW, _L), jnp.float32),
        jax.ShapeDtypeStruct((_NW, _L), jnp.float32),
    ),
    scratch_types=[
        pltpu.VMEM((_BPW,), jnp.int32),
        pltpu.VMEM((_BPW, _D), jnp.float32),
        pltpu.VMEM((_BPW, _D), jnp.float32),
        pltpu.VMEM((3, _L), jnp.float32),
        pltpu.SemaphoreType.DMA,
        pltpu.SemaphoreType.DMA,
    ],
)
def _sc_partials(idx_hbm, feat_hbm, table_hbm, e2_hbm, dt_hbm, f2_hbm,
                 idx_v, rows_v, feat_v, acc_v, gsem, fsem):
    wid = lax.axis_index("s") * _NC + lax.axis_index("c")
    base = wid * _BPW
    pltpu.sync_copy(idx_hbm.at[pl.ds(base, _BPW)], idx_v)
    fcopy = pltpu.async_copy(feat_hbm.at[pl.ds(base, _BPW)], feat_v, fsem)

    def fire(k, carry):
        iv = idx_v[pl.ds(k * _L, _L)]
        for j in range(_L):
            di = iv[j]
            pltpu.async_copy(table_hbm.at[pl.ds(di, 1)],
                             rows_v.at[pl.ds(k * _L + j, 1)], gsem)
        return carry

    lax.fori_loop(0, _BPW // _L, fire, 0)
    pltpu.make_async_copy(table_hbm.at[pl.ds(0, _BPW)], rows_v, gsem).wait()
    fcopy.wait()

    zeros = jnp.zeros((_L,), jnp.float32)

    def body(i, carry):
        e2, dt, f2 = carry
        for j in range(_D // _L):
            r = rows_v[i, pl.ds(j * _L, _L)]
            f = feat_v[i, pl.ds(j * _L, _L)]
            e2 = e2 + r * r
            dt = dt + r * f
            f2 = f2 + f * f
        return (e2, dt, f2)

    e2, dt, f2 = lax.fori_loop(0, _BPW, body, (zeros, zeros, zeros))
    acc_v[0, :] = e2
    acc_v[1, :] = dt
    acc_v[2, :] = f2
    pltpu.sync_copy(acc_v.at[0], e2_hbm.at[wid])
    pltpu.sync_copy(acc_v.at[1], dt_hbm.at[wid])
    pltpu.sync_copy(acc_v.at[2], f2_hbm.at[wid])


def _combine_body(e2_ref, dt_ref, f2_ref, o_ref):
    e2 = jnp.sum(e2_ref[...])
    dt = jnp.sum(dt_ref[...])
    f2 = jnp.sum(f2_ref[...])
    o_ref[0, 0] = 2.0 * f2 - 2.0 * jnp.sqrt(f2 / e2) * dt


_combine = pl.pallas_call(
    _combine_body,
    out_shape=jax.ShapeDtypeStruct((1, 1), jnp.float32),
    out_specs=pl.BlockSpec(memory_space=pltpu.SMEM),
)


def kernel(feature, item, neigh, pca_emb):
    idx = jnp.concatenate([item, neigh]).astype(jnp.int32)
    e2p, dtp, f2p = _sc_partials(idx, feature, pca_emb)
    out = _combine(e2p, dtp, f2p)
    return out[0, 0]


# speedup vs baseline: 1.4043x; 1.0053x over previous
import functools

import jax
import jax.numpy as jnp
from jax import lax
from jax.experimental import pallas as pl
from jax.experimental.pallas import tpu as pltpu
from jax.experimental.pallas import tpu_sc as plsc

_NC = 2
_NS = 16
_NW = _NC * _NS
_L = 16
_B = 8192
_D = 64
_BPW = _B // _NW

_mesh = plsc.VectorSubcoreMesh(core_axis_name="c", subcore_axis_name="s")


@functools.partial(
    pl.kernel,
    mesh=_mesh,
    compiler_params=pltpu.CompilerParams(use_tc_tiling_on_sc=True),
    out_type=(
        jax.ShapeDtypeStruct((_NW, _L), jnp.float32),
        jax.ShapeDtypeStruct((_NW, _L), jnp.float32),
        jax.ShapeDtypeStruct((_NW, _L), jnp.float32),
    ),
    scratch_types=[
        pltpu.VMEM((_BPW,), jnp.int32),
        pltpu.VMEM((_BPW, _D), jnp.float32),
        pltpu.VMEM((_BPW, _D), jnp.float32),
        pltpu.VMEM((3, _L), jnp.float32),
        pltpu.SemaphoreType.DMA,
        pltpu.SemaphoreType.DMA,
    ],
)
def _sc_partials(idx_hbm, feat_hbm, table_hbm, e2_hbm, dt_hbm, f2_hbm,
                 idx_v, rows_v, feat_v, acc_v, gsem, fsem):
    wid = lax.axis_index("s") * _NC + lax.axis_index("c")
    base = wid * _BPW
    pltpu.sync_copy(idx_hbm.at[pl.ds(base, _BPW)], idx_v)
    fcopy = pltpu.async_copy(feat_hbm.at[pl.ds(base, _BPW)], feat_v, fsem)

    def fire(k, carry):
        iv = idx_v[pl.ds(k * _L, _L)]
        for j in range(_L):
            di = iv[j]
            pltpu.async_copy(table_hbm.at[pl.ds(di, 1)],
                             rows_v.at[pl.ds(k * _L + j, 1)], gsem)
        return carry

    lax.fori_loop(0, _BPW // _L, fire, 0)
    pltpu.make_async_copy(table_hbm.at[pl.ds(0, _BPW)], rows_v, gsem).wait()
    fcopy.wait()

    zeros = jnp.zeros((_L,), jnp.float32)

    def body(i, carry):
        e2, dt, f2 = carry
        for j in range(_D // _L):
            r = rows_v[i, pl.ds(j * _L, _L)]
            f = feat_v[i, pl.ds(j * _L, _L)]
            e2 = e2 + r * r
            dt = dt + r * f
            f2 = f2 + f * f
        return (e2, dt, f2)

    e2, dt, f2 = lax.fori_loop(0, _BPW, body, (zeros, zeros, zeros))
    acc_v[0, :] = e2
    acc_v[1, :] = dt
    acc_v[2, :] = f2
    pltpu.sync_copy(acc_v.at[0], e2_hbm.at[wid])
    pltpu.sync_copy(acc_v.at[1], dt_hbm.at[wid])
    pltpu.sync_copy(acc_v.at[2], f2_hbm.at[wid])


def _combine_body(e2_ref, dt_ref, f2_ref, o_ref):
    e2 = jnp.sum(e2_ref[...])
    dt = jnp.sum(dt_ref[...])
    f2 = jnp.sum(f2_ref[...])
    o_ref[0, 0] = 2.0 * f2 - 2.0 * jnp.sqrt(f2 / e2) * dt


_combine = pl.pallas_call(
    _combine_body,
    out_shape=jax.ShapeDtypeStruct((1, 1), jnp.float32),
    out_specs=pl.BlockSpec(memory_space=pltpu.SMEM),
)


def kernel(feature, item, neigh, pca_emb):
    idx = jnp.concatenate([item, neigh]).astype(jnp.int32)
    e2p, dtp, f2p = _sc_partials(idx, feature, pca_emb)
    out = _combine(e2p, dtp, f2p)
    return out[0, 0]
